# baseline (speedup 1.0000x reference)
import jax
import jax.numpy as jnp
import numpy as np
from jax.experimental import pallas as pl
from jax.experimental.pallas import tpu as pltpu

PAD_IDX = 0
NEG = -1e9
D, H, DH, L = 16, 2, 8, 50
R = 8
NROW = 2 * R
ST = 56
RSQ = 1.0 / np.sqrt(DH)


def _make_body(nb, m):
    def body(idxcat_ref, emb3_ref, wpad_ref, bfull_ref, ubp_ref, ibp_ref,
             gbv_ref, o_ref, gbuf, qs, ab, idxs, sem):
        i = pl.program_id(0)
        slot = jax.lax.rem(i, 2)

        @pl.when(i == 0)
        def _():
            pltpu.make_async_copy(idxcat_ref.at[0, 0], idxs.at[0],
                                  sem.at[0]).start()

        @pl.when(i + 1 < nb)
        def _():
            nslot = jax.lax.rem(i + 1, 2)
            pltpu.make_async_copy(idxcat_ref.at[i + 1, 0], idxs.at[nslot],
                                  sem.at[nslot]).start()

        pltpu.make_async_copy(idxcat_ref.at[i, 0], idxs.at[slot],
                              sem.at[slot]).wait()

        for mi in range(2 * m):
            w = idxs[slot, 2 * m + mi]
            sh = idxs[slot, 4 * m + mi]
            gbuf[mi, :] = pltpu.roll(emb3_ref[w], sh, axis=1)[0, :]

        qkvp = jnp.dot(gbuf[...], wpad_ref[...],
                       preferred_element_type=jnp.float32) + bfull_ref[...]
        for r in range(NROW):
            qs[r * ST:r * ST + L, 0:66] = qkvp[r * L:(r + 1) * L, :]

        idxv = idxcat_ref[i]

        lm = jax.lax.broadcasted_iota(jnp.int32, (1, 16), 1) < 8
        for r in range(NROW):
            b = r * ST
            q = qs[b:b + L, 0:16]
            kk = qs[b:b + L, 16:32]
            kcat = jnp.concatenate(
                [jnp.where(lm, kk, 0.0), jnp.where(lm, 0.0, kk)], axis=0)
            logits = jax.lax.dot_general(
                q, kcat, (((1,), (1,)), ((), ())),
                preferred_element_type=jnp.float32) * RSQ
            pen = jnp.where(idxv[0:1, r * L:(r + 1) * L] != PAD_IDX, 0.0, NEG)
            ab[b:b + L, 0:2 * L] = logits + jnp.concatenate([pen, pen],
                                                            axis=1)

        nchunk = 4
        rows = NROW * ST // nchunk
        for c in range(nchunk):
            sl = slice(c * rows, (c + 1) * rows)
            lg = ab[sl, 0:2 * L]
            l0 = lg[:, 0:L]
            l1 = lg[:, L:2 * L]
            e0 = jnp.exp(l0 - jnp.max(l0, axis=1, keepdims=True))
            a0 = e0 / jnp.sum(e0, axis=1, keepdims=True)
            e1 = jnp.exp(l1 - jnp.max(l1, axis=1, keepdims=True))
            a1 = e1 / jnp.sum(e1, axis=1, keepdims=True)
            ab[sl, 0:2 * L] = jnp.concatenate([a0, a1], axis=1)

        pooled = []
        for r in range(NROW):
            b = r * ST
            acat = ab[b:b + L, 0:2 * L]
            a0 = acat[:, 0:L]
            a1 = acat[:, L:2 * L]
            scores = (
                jnp.dot(a0, qs[b:b + L, 64:65],
                        preferred_element_type=jnp.float32) +
                jnp.dot(a1, qs[b:b + L, 65:66],
                        preferred_element_type=jnp.float32))
            mrow = idxv[0:1, r * L:(r + 1) * L] != PAD_IDX
            nvalid = jnp.sum(mrow.astype(jnp.int32), axis=1, keepdims=True)
            qmask = jax.lax.broadcasted_iota(jnp.int32, (L, 1), 0) < nvalid
            scores = jnp.where(qmask, scores, NEG)
            we = jnp.exp(scores - jnp.max(scores, axis=0, keepdims=True))
            wv = we / jnp.sum(we, axis=0, keepdims=True)
            av = (jnp.dot(a0, qs[b:b + L, 32:48],
                          preferred_element_type=jnp.float32) +
                  jnp.dot(a1, qs[b:b + L, 48:64],
                          preferred_element_type=jnp.float32))
            pooled.append(jnp.sum(av * wv, axis=0, keepdims=True))

        iota128 = jax.lax.broadcasted_iota(jnp.int32, (1, 128), 1)
        vals = []
        for r in range(R):
            dot = jnp.sum(pooled[r] * pooled[R + r], axis=1, keepdims=True)
            urow = ubp_ref[idxs[slot, 6 * m + r]]
            ub = jnp.sum(jnp.where(iota128 == idxs[slot, 6 * m + R + r],
                                   urow, 0.0), axis=1, keepdims=True)
            irow = ibp_ref[idxs[slot, 6 * m + 2 * R + r]]
            ib = jnp.sum(jnp.where(iota128 == idxs[slot, 6 * m + 3 * R + r],
                                   irow, 0.0), axis=1, keepdims=True)
            vals.append(dot + ub + ib)
        out = jnp.concatenate(vals, axis=1) + gbv_ref[0:1, 0:R]
        o_ref[...] = out.reshape(1, 1, R)
    return body


def kernel(subject_emb, in_proj_w, in_proj_b, out_w, out_b, attn_weight,
           attn_bias, user_bias, item_bias, global_bias,
           user_idx, item_idx, fav_subjects, book_subjects):
    B = fav_subjects.shape[0]
    NB = B // R
    M = R * L
    NS = subject_emb.shape[0]

    Wq, Wk, Wv = in_proj_w[0:16], in_proj_w[16:32], in_proj_w[32:48]
    bq, bk, bv = in_proj_b[0:16], in_proj_b[16:32], in_proj_b[32:48]
    Wo0, Wo1 = out_w[:, 0:8], out_w[:, 8:16]
    Wvp0 = (Wo0 @ Wv[0:8]).T
    Wvp1 = (Wo1 @ Wv[8:16]).T
    bvp0 = bv[0:8] @ Wo0.T + out_b / 2.0
    bvp1 = bv[8:16] @ Wo1.T + out_b / 2.0
    Wcat = jnp.concatenate([Wq.T, Wk.T, Wvp0, Wvp1], axis=1)
    bcat = jnp.concatenate([bq, bk, bvp0, bvp1])
    ws0 = Wvp0 @ attn_weight
    ws1 = Wvp1 @ attn_weight
    bs0 = (bvp0 @ attn_weight + jnp.sum(attn_bias) / 2.0)[None]
    bs1 = (bvp1 @ attn_weight + jnp.sum(attn_bias) / 2.0)[None]
    Wfull = jnp.concatenate([Wcat, ws0[:, None], ws1[:, None]], axis=1)
    bfull = jnp.concatenate([bcat, bs0, bs1])[None, :]
    Wpad = jnp.zeros((128, 66), jnp.float32).at[0:16].set(Wfull)

    subj = jnp.concatenate([fav_subjects.reshape(NB, M),
                            book_subjects.reshape(NB, M)],
                           axis=1).astype(jnp.int32)
    word = subj >> 3
    shift = ((8 - (subj & 7)) & 7) << 4
    u32 = user_idx.reshape(NB, R).astype(jnp.int32)
    i32_ = item_idx.reshape(NB, R).astype(jnp.int32)
    idxcat = jnp.concatenate(
        [subj, word, shift, u32 >> 7, u32 & 127, i32_ >> 7, i32_ & 127],
        axis=1)
    lanes = ((idxcat.shape[1] + 127) // 128) * 128
    idxcat = jnp.pad(idxcat, ((0, 0), (0, lanes - idxcat.shape[1])))
    idxcat = idxcat.reshape(NB, 1, lanes)

    emb3 = subject_emb.reshape(NS // 8, 1, 128)
    nub = (user_bias.shape[0] + 127) // 128
    ubp = jnp.pad(user_bias[:, 0],
                  (0, nub * 128 - user_bias.shape[0])).reshape(nub, 1, 128)
    nib = (item_bias.shape[0] + 127) // 128
    ibp = jnp.pad(item_bias[:, 0],
                  (0, nib * 128 - item_bias.shape[0])).reshape(nib, 1, 128)
    gbv = jnp.broadcast_to(global_bias.reshape(1, 1), (1, 128))

    out = pl.pallas_call(
        _make_body(NB, M),
        out_shape=jax.ShapeDtypeStruct((NB, 1, R), jnp.float32),
        grid=(NB,),
        in_specs=[
            pl.BlockSpec((NB, 1, lanes), lambda i: (0, 0, 0)),
            pl.BlockSpec((NS // 8, 1, 128), lambda i: (0, 0, 0)),
            pl.BlockSpec((128, 66), lambda i: (0, 0)),
            pl.BlockSpec((1, 66), lambda i: (0, 0)),
            pl.BlockSpec((nub, 1, 128), lambda i: (0, 0, 0)),
            pl.BlockSpec((nib, 1, 128), lambda i: (0, 0, 0)),
            pl.BlockSpec((1, 128), lambda i: (0, 0)),
        ],
        out_specs=pl.BlockSpec((1, 1, R), lambda i: (i, 0, 0)),
        scratch_shapes=[
            pltpu.VMEM((2 * M, 128), jnp.float32),
            pltpu.VMEM((NROW * ST, 128), jnp.float32),
            pltpu.VMEM((NROW * ST, 128), jnp.float32),
            pltpu.SMEM((2, lanes), jnp.int32),
            pltpu.SemaphoreType.DMA((2,)),
        ],
        compiler_params=pltpu.CompilerParams(
            dimension_semantics=("parallel",),
            vmem_limit_bytes=50 * 1024 * 1024,
        ),
        name="fused_pooler",
    )(idxcat, emb3, Wpad, bfull, ubp, ibp, gbv)
    return out.reshape(B)


# speedup vs baseline: 1.0422x; 1.0422x over previous
import jax
import jax.numpy as jnp
import numpy as np
from jax.experimental import pallas as pl
from jax.experimental.pallas import tpu as pltpu

PAD_IDX = 0
NEG = -1e9
D, H, DH, L = 16, 2, 8, 50
R = 16
NROW = 2 * R
ST = 56
RSQ = 1.0 / np.sqrt(DH)


def _make_body(nb, m):
    def body(idxcat_ref, emb3_ref, wpad_ref, bfull_ref, ubp_ref, ibp_ref,
             gbv_ref, o_ref, gbuf, qs, ab, idxs, sem):
        i = pl.program_id(0)
        slot = jax.lax.rem(i, 2)

        @pl.when(i == 0)
        def _():
            pltpu.make_async_copy(idxcat_ref.at[0, 0], idxs.at[0],
                                  sem.at[0]).start()

        @pl.when(i + 1 < nb)
        def _():
            nslot = jax.lax.rem(i + 1, 2)
            pltpu.make_async_copy(idxcat_ref.at[i + 1, 0], idxs.at[nslot],
                                  sem.at[nslot]).start()

        pltpu.make_async_copy(idxcat_ref.at[i, 0], idxs.at[slot],
                              sem.at[slot]).wait()

        for mi in range(2 * m):
            w = idxs[slot, 2 * m + mi]
            sh = idxs[slot, 4 * m + mi]
            gbuf[mi, :] = pltpu.roll(emb3_ref[w], sh, axis=1)[0, :]

        qkvp = jnp.dot(gbuf[...], wpad_ref[...],
                       preferred_element_type=jnp.float32) + bfull_ref[...]
        for r in range(NROW):
            qs[r * ST:r * ST + L, 0:66] = qkvp[r * L:(r + 1) * L, :]

        idxv = idxcat_ref[i]

        lm = jax.lax.broadcasted_iota(jnp.int32, (1, 16), 1) < 8
        for r in range(NROW):
            b = r * ST
            q = qs[b:b + L, 0:16]
            kk = qs[b:b + L, 16:32]
            kcat = jnp.concatenate(
                [jnp.where(lm, kk, 0.0), jnp.where(lm, 0.0, kk)], axis=0)
            logits = jax.lax.dot_general(
                q, kcat, (((1,), (1,)), ((), ())),
                preferred_element_type=jnp.float32) * RSQ
            pen = jnp.where(idxv[0:1, r * L:(r + 1) * L] != PAD_IDX, 0.0, NEG)
            ab[b:b + L, 0:2 * L] = logits + jnp.concatenate([pen, pen],
                                                            axis=1)

        nchunk = 8
        rows = NROW * ST // nchunk
        for c in range(nchunk):
            sl = slice(c * rows, (c + 1) * rows)
            lg = ab[sl, 0:2 * L]
            l0 = lg[:, 0:L]
            l1 = lg[:, L:2 * L]
            e0 = jnp.exp(l0 - jnp.max(l0, axis=1, keepdims=True))
            a0 = e0 / jnp.sum(e0, axis=1, keepdims=True)
            e1 = jnp.exp(l1 - jnp.max(l1, axis=1, keepdims=True))
            a1 = e1 / jnp.sum(e1, axis=1, keepdims=True)
            ab[sl, 0:2 * L] = jnp.concatenate([a0, a1], axis=1)

        pooled = []
        for r in range(NROW):
            b = r * ST
            acat = ab[b:b + L, 0:2 * L]
            a0 = acat[:, 0:L]
            a1 = acat[:, L:2 * L]
            scores = (
                jnp.dot(a0, qs[b:b + L, 64:65],
                        preferred_element_type=jnp.float32) +
                jnp.dot(a1, qs[b:b + L, 65:66],
                        preferred_element_type=jnp.float32))
            mrow = idxv[0:1, r * L:(r + 1) * L] != PAD_IDX
            nvalid = jnp.sum(mrow.astype(jnp.int32), axis=1, keepdims=True)
            qmask = jax.lax.broadcasted_iota(jnp.int32, (L, 1), 0) < nvalid
            scores = jnp.where(qmask, scores, NEG)
            we = jnp.exp(scores - jnp.max(scores, axis=0, keepdims=True))
            wv = we / jnp.sum(we, axis=0, keepdims=True)
            av = (jnp.dot(a0, qs[b:b + L, 32:48],
                          preferred_element_type=jnp.float32) +
                  jnp.dot(a1, qs[b:b + L, 48:64],
                          preferred_element_type=jnp.float32))
            pooled.append(jnp.sum(av * wv, axis=0, keepdims=True))

        iota128 = jax.lax.broadcasted_iota(jnp.int32, (1, 128), 1)
        vals = []
        for r in range(R):
            dot = jnp.sum(pooled[r] * pooled[R + r], axis=1, keepdims=True)
            urow = ubp_ref[idxs[slot, 6 * m + r]]
            ub = jnp.sum(jnp.where(iota128 == idxs[slot, 6 * m + R + r],
                                   urow, 0.0), axis=1, keepdims=True)
            irow = ibp_ref[idxs[slot, 6 * m + 2 * R + r]]
            ib = jnp.sum(jnp.where(iota128 == idxs[slot, 6 * m + 3 * R + r],
                                   irow, 0.0), axis=1, keepdims=True)
            vals.append(dot + ub + ib)
        out = jnp.concatenate(vals, axis=1) + gbv_ref[0:1, 0:R]
        o_ref[...] = out.reshape(1, 1, R)
    return body


def kernel(subject_emb, in_proj_w, in_proj_b, out_w, out_b, attn_weight,
           attn_bias, user_bias, item_bias, global_bias,
           user_idx, item_idx, fav_subjects, book_subjects):
    B = fav_subjects.shape[0]
    NB = B // R
    M = R * L
    NS = subject_emb.shape[0]

    Wq, Wk, Wv = in_proj_w[0:16], in_proj_w[16:32], in_proj_w[32:48]
    bq, bk, bv = in_proj_b[0:16], in_proj_b[16:32], in_proj_b[32:48]
    Wo0, Wo1 = out_w[:, 0:8], out_w[:, 8:16]
    Wvp0 = (Wo0 @ Wv[0:8]).T
    Wvp1 = (Wo1 @ Wv[8:16]).T
    bvp0 = bv[0:8] @ Wo0.T + out_b / 2.0
    bvp1 = bv[8:16] @ Wo1.T + out_b / 2.0
    Wcat = jnp.concatenate([Wq.T, Wk.T, Wvp0, Wvp1], axis=1)
    bcat = jnp.concatenate([bq, bk, bvp0, bvp1])
    ws0 = Wvp0 @ attn_weight
    ws1 = Wvp1 @ attn_weight
    bs0 = (bvp0 @ attn_weight + jnp.sum(attn_bias) / 2.0)[None]
    bs1 = (bvp1 @ attn_weight + jnp.sum(attn_bias) / 2.0)[None]
    Wfull = jnp.concatenate([Wcat, ws0[:, None], ws1[:, None]], axis=1)
    bfull = jnp.concatenate([bcat, bs0, bs1])[None, :]
    Wpad = jnp.zeros((128, 66), jnp.float32).at[0:16].set(Wfull)

    subj = jnp.concatenate([fav_subjects.reshape(NB, M),
                            book_subjects.reshape(NB, M)],
                           axis=1).astype(jnp.int32)
    word = subj >> 3
    shift = ((8 - (subj & 7)) & 7) << 4
    u32 = user_idx.reshape(NB, R).astype(jnp.int32)
    i32_ = item_idx.reshape(NB, R).astype(jnp.int32)
    idxcat = jnp.concatenate(
        [subj, word, shift, u32 >> 7, u32 & 127, i32_ >> 7, i32_ & 127],
        axis=1)
    lanes = ((idxcat.shape[1] + 127) // 128) * 128
    idxcat = jnp.pad(idxcat, ((0, 0), (0, lanes - idxcat.shape[1])))
    idxcat = idxcat.reshape(NB, 1, lanes)

    emb3 = subject_emb.reshape(NS // 8, 1, 128)
    nub = (user_bias.shape[0] + 127) // 128
    ubp = jnp.pad(user_bias[:, 0],
                  (0, nub * 128 - user_bias.shape[0])).reshape(nub, 1, 128)
    nib = (item_bias.shape[0] + 127) // 128
    ibp = jnp.pad(item_bias[:, 0],
                  (0, nib * 128 - item_bias.shape[0])).reshape(nib, 1, 128)
    gbv = jnp.broadcast_to(global_bias.reshape(1, 1), (1, 128))

    out = pl.pallas_call(
        _make_body(NB, M),
        out_shape=jax.ShapeDtypeStruct((NB, 1, R), jnp.float32),
        grid=(NB,),
        in_specs=[
            pl.BlockSpec((NB, 1, lanes), lambda i: (0, 0, 0)),
            pl.BlockSpec((NS // 8, 1, 128), lambda i: (0, 0, 0)),
            pl.BlockSpec((128, 66), lambda i: (0, 0)),
            pl.BlockSpec((1, 66), lambda i: (0, 0)),
            pl.BlockSpec((nub, 1, 128), lambda i: (0, 0, 0)),
            pl.BlockSpec((nib, 1, 128), lambda i: (0, 0, 0)),
            pl.BlockSpec((1, 128), lambda i: (0, 0)),
        ],
        out_specs=pl.BlockSpec((1, 1, R), lambda i: (i, 0, 0)),
        scratch_shapes=[
            pltpu.VMEM((2 * M, 128), jnp.float32),
            pltpu.VMEM((NROW * ST, 128), jnp.float32),
            pltpu.VMEM((NROW * ST, 128), jnp.float32),
            pltpu.SMEM((2, lanes), jnp.int32),
            pltpu.SemaphoreType.DMA((2,)),
        ],
        compiler_params=pltpu.CompilerParams(
            dimension_semantics=("parallel",),
            vmem_limit_bytes=50 * 1024 * 1024,
        ),
        name="fused_pooler",
    )(idxcat, emb3, Wpad, bfull, ubp, ibp, gbv)
    return out.reshape(B)
